# baseline (speedup 1.0000x reference)
import functools

import jax
import jax.numpy as jnp
from jax.experimental import pallas as pl
from jax.experimental.pallas import tpu as pltpu

_BLK = 1024


def _body(gid_ref, x_ref, w1_ref, b1_ref, w2_ref, b2_ref, o_ref):
    x = x_ref[...]
    g = gid_ref[...]
    E = w1_ref.shape[0]
    acc = None
    for e in range(E):
        h = jnp.dot(x, w1_ref[e], preferred_element_type=jnp.float32)
        h = jnp.maximum(h + b1_ref[e], 0.0)
        oe = jnp.dot(h, w2_ref[e], preferred_element_type=jnp.float32)
        oe = oe + b2_ref[e]
        sel = jnp.where(g == e, oe, 0.0)
        acc = sel if acc is None else acc + sel
    o_ref[...] = acc


def kernel(obs, group_ids, W1, b1, W2, b2, ph_to_feature):
    n_threads, n_agents, d = obs.shape
    E, dp1, H = W1.shape
    A = W2.shape[2]
    T = n_threads * n_agents
    nb = T // _BLK

    x = obs.reshape(T, d)
    gid2 = group_ids.reshape(T, 1)
    W1m = W1[:, :d, :]
    b1eff = b1 + ph_to_feature * W1[:, d, :]

    out = pl.pallas_call(
        _body,
        grid=(nb,),
        in_specs=[
            pl.BlockSpec((_BLK, 1), lambda i: (i, 0)),
            pl.BlockSpec((_BLK, d), lambda i: (i, 0)),
            pl.BlockSpec((E, d, H), lambda i: (0, 0, 0)),
            pl.BlockSpec((E, H), lambda i: (0, 0)),
            pl.BlockSpec((E, H, A), lambda i: (0, 0, 0)),
            pl.BlockSpec((E, A), lambda i: (0, 0)),
        ],
        out_specs=pl.BlockSpec((_BLK, A), lambda i: (i, 0)),
        out_shape=jax.ShapeDtypeStruct((T, A), jnp.float32),
        compiler_params=pltpu.CompilerParams(
            dimension_semantics=("arbitrary",),
        ),
    )(gid2, x, W1m, b1eff, W2, b2)
    return out.reshape(n_threads, n_agents, A)


# speedup vs baseline: 1.0100x; 1.0100x over previous
import jax
import jax.numpy as jnp
from jax.experimental import pallas as pl
from jax.experimental.pallas import tpu as pltpu

_BLK = 1024


def _body(gid_ref, x_ref, w1_ref, b1_ref, w2_ref, b2_ref, o_ref):
    B = x_ref.shape[0]
    EH = w1_ref.shape[1]
    H = EH // b2_ref.shape[0]
    A = w2_ref.shape[1]

    x = x_ref[...].astype(jnp.bfloat16)
    w1 = w1_ref[...].astype(jnp.bfloat16)
    h = jnp.dot(x, w1, preferred_element_type=jnp.float32)
    h = jnp.maximum(h + b1_ref[...], 0.0)

    g = gid_ref[...]
    col_expert = jax.lax.broadcasted_iota(jnp.int32, (1, EH), 1) // H
    h = jnp.where(g == col_expert, h, 0.0)

    w2 = w2_ref[...].astype(jnp.bfloat16)
    out = jnp.dot(h.astype(jnp.bfloat16), w2,
                  preferred_element_type=jnp.float32)

    E = b2_ref.shape[0]
    b2sel = jnp.zeros((B, A), jnp.float32)
    for e in range(E):
        b2sel = jnp.where(g == e, b2_ref[e], b2sel)
    o_ref[...] = out + b2sel


def kernel(obs, group_ids, W1, b1, W2, b2, ph_to_feature):
    n_threads, n_agents, d = obs.shape
    E, dp1, H = W1.shape
    A = W2.shape[2]
    T = n_threads * n_agents
    nb = T // _BLK

    x = obs.reshape(T, d)
    gid2 = group_ids.reshape(T, 1)
    b1eff = (b1 + ph_to_feature * W1[:, d, :]).reshape(1, E * H)
    W1all = jnp.transpose(W1[:, :d, :], (1, 0, 2)).reshape(d, E * H)
    W2all = W2.reshape(E * H, A)

    out = pl.pallas_call(
        _body,
        grid=(nb,),
        in_specs=[
            pl.BlockSpec((_BLK, 1), lambda i: (i, 0)),
            pl.BlockSpec((_BLK, d), lambda i: (i, 0)),
            pl.BlockSpec((d, E * H), lambda i: (0, 0)),
            pl.BlockSpec((1, E * H), lambda i: (0, 0)),
            pl.BlockSpec((E * H, A), lambda i: (0, 0)),
            pl.BlockSpec((E, A), lambda i: (0, 0)),
        ],
        out_specs=pl.BlockSpec((_BLK, A), lambda i: (i, 0)),
        out_shape=jax.ShapeDtypeStruct((T, A), jnp.float32),
        compiler_params=pltpu.CompilerParams(
            dimension_semantics=("arbitrary",),
        ),
    )(gid2, x, W1all, b1eff, W2all, b2)
    return out.reshape(n_threads, n_agents, A)


# speedup vs baseline: 1.1159x; 1.1049x over previous
import jax
import jax.numpy as jnp
from jax.experimental import pallas as pl
from jax.experimental.pallas import tpu as pltpu

_BLK = 1024


def _body(gid_ref, x_ref, w1_ref, b1_ref, w2_ref, b2_ref, exp_ref, o_ref):
    EH = w1_ref.shape[1]
    E = b2_ref.shape[0]

    x = x_ref[...].astype(jnp.bfloat16)
    w1 = w1_ref[...]
    h = jnp.dot(x, w1, preferred_element_type=jnp.float32)
    h = jnp.maximum(h + b1_ref[...], 0.0)

    g = gid_ref[...]
    eids = jax.lax.broadcasted_iota(jnp.int32, (1, E), 1)
    onehot = jnp.where(g == eids, 1.0, 0.0).astype(jnp.bfloat16)
    mask = jnp.dot(onehot, exp_ref[...],
                   preferred_element_type=jnp.float32)
    b2sel = jnp.dot(onehot, b2_ref[...].astype(jnp.bfloat16),
                    preferred_element_type=jnp.float32)

    h = h * mask
    out = jnp.dot(h, w2_ref[...], preferred_element_type=jnp.float32)
    o_ref[...] = out + b2sel


def kernel(obs, group_ids, W1, b1, W2, b2, ph_to_feature):
    n_threads, n_agents, d = obs.shape
    E, dp1, H = W1.shape
    A = W2.shape[2]
    T = n_threads * n_agents
    nb = T // _BLK

    x = obs.reshape(T, d)
    gid2 = group_ids.reshape(T, 1)
    b1eff = (b1 + ph_to_feature * W1[:, d, :]).reshape(1, E * H)
    W1all = jnp.transpose(W1[:, :d, :], (1, 0, 2)).reshape(d, E * H)
    W1all = W1all.astype(jnp.bfloat16)
    W2all = W2.reshape(E * H, A)
    expander = jnp.repeat(jnp.eye(E, dtype=jnp.bfloat16), H, axis=1)

    out = pl.pallas_call(
        _body,
        grid=(nb,),
        in_specs=[
            pl.BlockSpec((_BLK, 1), lambda i: (i, 0)),
            pl.BlockSpec((_BLK, d), lambda i: (i, 0)),
            pl.BlockSpec((d, E * H), lambda i: (0, 0)),
            pl.BlockSpec((1, E * H), lambda i: (0, 0)),
            pl.BlockSpec((E * H, A), lambda i: (0, 0)),
            pl.BlockSpec((E, A), lambda i: (0, 0)),
            pl.BlockSpec((E, E * H), lambda i: (0, 0)),
        ],
        out_specs=pl.BlockSpec((_BLK, A), lambda i: (i, 0)),
        out_shape=jax.ShapeDtypeStruct((T, A), jnp.float32),
        compiler_params=pltpu.CompilerParams(
            dimension_semantics=("arbitrary",),
        ),
    )(gid2, x, W1all, b1eff, W2all, b2, expander)
    return out.reshape(n_threads, n_agents, A)


# speedup vs baseline: 1.1690x; 1.0476x over previous
import jax
import jax.numpy as jnp
from jax.experimental import pallas as pl
from jax.experimental.pallas import tpu as pltpu

_BLK = 2048


def _body(gid_ref, x_ref, w1_hbm, b1_hbm, w2_hbm, b2_hbm, exp_hbm, o_ref,
          w1_v, b1_v, w2_v, b2_v, exp_v, sem):
    i = pl.program_id(0)

    @pl.when(i == 0)
    def _load_weights():
        for src, dst in ((w1_hbm, w1_v), (b1_hbm, b1_v), (w2_hbm, w2_v),
                         (b2_hbm, b2_v), (exp_hbm, exp_v)):
            cp = pltpu.make_async_copy(src, dst, sem)
            cp.start()
            cp.wait()

    E = b2_v.shape[0]
    x = x_ref[...].astype(jnp.bfloat16)
    h = jnp.dot(x, w1_v[...], preferred_element_type=jnp.float32)
    h = jnp.maximum(h + b1_v[...], 0.0)

    g = gid_ref[...]
    eids = jax.lax.broadcasted_iota(jnp.int32, (1, E), 1)
    onehot = jnp.where(g == eids, 1.0, 0.0).astype(jnp.bfloat16)
    mask = jnp.dot(onehot, exp_v[...],
                   preferred_element_type=jnp.float32)
    b2sel = jnp.dot(onehot, b2_v[...],
                    preferred_element_type=jnp.float32)

    hb = (h * mask).astype(jnp.bfloat16)
    out = jnp.dot(hb, w2_v[...], preferred_element_type=jnp.float32)
    o_ref[...] = out + b2sel


def kernel(obs, group_ids, W1, b1, W2, b2, ph_to_feature):
    n_threads, n_agents, d = obs.shape
    E, dp1, H = W1.shape
    A = W2.shape[2]
    T = n_threads * n_agents
    nb = T // _BLK
    EH = E * H

    x = obs.reshape(T, d)
    gid2 = group_ids.reshape(T, 1)
    b1eff = (b1 + ph_to_feature * W1[:, d, :]).reshape(1, EH)
    W1all = jnp.transpose(W1[:, :d, :], (1, 0, 2)).reshape(d, EH)
    W1all = W1all.astype(jnp.bfloat16)
    W2all = W2.reshape(EH, A).astype(jnp.bfloat16)
    b2b = b2.astype(jnp.bfloat16)
    expander = jnp.repeat(jnp.eye(E, dtype=jnp.bfloat16), H, axis=1)

    any_spec = pl.BlockSpec(memory_space=pl.ANY)
    out = pl.pallas_call(
        _body,
        grid=(nb,),
        in_specs=[
            pl.BlockSpec((_BLK, 1), lambda i: (i, 0)),
            pl.BlockSpec((_BLK, d), lambda i: (i, 0)),
            any_spec, any_spec, any_spec, any_spec, any_spec,
        ],
        out_specs=pl.BlockSpec((_BLK, A), lambda i: (i, 0)),
        out_shape=jax.ShapeDtypeStruct((T, A), jnp.float32),
        scratch_shapes=[
            pltpu.VMEM((d, EH), jnp.bfloat16),
            pltpu.VMEM((1, EH), jnp.float32),
            pltpu.VMEM((EH, A), jnp.bfloat16),
            pltpu.VMEM((E, A), jnp.bfloat16),
            pltpu.VMEM((E, EH), jnp.bfloat16),
            pltpu.SemaphoreType.DMA,
        ],
        compiler_params=pltpu.CompilerParams(
            dimension_semantics=("arbitrary",),
        ),
    )(gid2, x, W1all, b1eff, W2all, b2b, expander)
    return out.reshape(n_threads, n_agents, A)


# speedup vs baseline: 1.1761x; 1.0060x over previous
import jax
import jax.numpy as jnp
from jax.experimental import pallas as pl
from jax.experimental.pallas import tpu as pltpu

_BLK = 2048


def _body(gid_ref, x_ref, w1_hbm, w2_hbm, b2_hbm, exp_hbm, o_ref,
          w1_v, w2_v, b2_v, exp_v, sem):
    i = pl.program_id(0)

    @pl.when(i == 0)
    def _load_weights():
        for src, dst in ((w1_hbm, w1_v), (w2_hbm, w2_v),
                         (b2_hbm, b2_v), (exp_hbm, exp_v)):
            cp = pltpu.make_async_copy(src, dst, sem)
            cp.start()
            cp.wait()

    E = b2_v.shape[0]
    eids = jax.lax.broadcasted_iota(jnp.int32, (1, E), 1)
    g = gid_ref[...]
    onehot = jnp.where(g == eids, 1.0, 0.0).astype(jnp.bfloat16)
    SUB = 512
    for s in range(_BLK // SUB):
        sl = slice(s * SUB, (s + 1) * SUB)
        oh = onehot[sl, :]
        x = x_ref[sl, :].astype(jnp.bfloat16)
        hpre = jnp.dot(x, w1_v[...], preferred_element_type=jnp.float32)
        bm = jnp.dot(oh, exp_v[...], preferred_element_type=jnp.float32)
        h = jnp.maximum(hpre + bm, 0.0)
        b2sel = jnp.dot(oh, b2_v[...],
                        preferred_element_type=jnp.float32)
        hb = h.astype(jnp.bfloat16)
        out = jnp.dot(hb, w2_v[...], preferred_element_type=jnp.float32)
        o_ref[sl, :] = out + b2sel


def kernel(obs, group_ids, W1, b1, W2, b2, ph_to_feature):
    n_threads, n_agents, d = obs.shape
    E, dp1, H = W1.shape
    A = W2.shape[2]
    T = n_threads * n_agents
    nb = T // _BLK
    EH = E * H

    x = obs.reshape(T, d)
    gid2 = group_ids.reshape(T, 1)
    b1eff = (b1 + ph_to_feature * W1[:, d, :]).reshape(1, EH)
    W1all = jnp.transpose(W1[:, :d, :], (1, 0, 2)).reshape(d, EH)
    W1all = W1all.astype(jnp.bfloat16)
    W2all = W2.reshape(EH, A).astype(jnp.bfloat16)
    b2b = b2.astype(jnp.bfloat16)
    col_e = (jnp.arange(EH, dtype=jnp.int32) // H)[None, :]
    row_e = jnp.arange(E, dtype=jnp.int32)[:, None]
    biasmask = jnp.where(row_e == col_e, b1eff, -1e30).astype(jnp.bfloat16)

    any_spec = pl.BlockSpec(memory_space=pl.ANY)
    out = pl.pallas_call(
        _body,
        grid=(nb,),
        in_specs=[
            pl.BlockSpec((_BLK, 1), lambda i: (i, 0)),
            pl.BlockSpec((_BLK, d), lambda i: (i, 0)),
            any_spec, any_spec, any_spec, any_spec,
        ],
        out_specs=pl.BlockSpec((_BLK, A), lambda i: (i, 0)),
        out_shape=jax.ShapeDtypeStruct((T, A), jnp.float32),
        scratch_shapes=[
            pltpu.VMEM((d, EH), jnp.bfloat16),
            pltpu.VMEM((EH, A), jnp.bfloat16),
            pltpu.VMEM((E, A), jnp.bfloat16),
            pltpu.VMEM((E, EH), jnp.bfloat16),
            pltpu.SemaphoreType.DMA,
        ],
        compiler_params=pltpu.CompilerParams(
            dimension_semantics=("arbitrary",),
        ),
    )(gid2, x, W1all, W2all, b2b, biasmask)
    return out.reshape(n_threads, n_agents, A)


# speedup vs baseline: 1.1905x; 1.0123x over previous
import jax
import jax.numpy as jnp
from jax.experimental import pallas as pl
from jax.experimental.pallas import tpu as pltpu

_BLK = 4096


def _body(gid_ref, x_ref, w1_hbm, w2_hbm, b2_hbm, exp_hbm, o_ref,
          w1_v, w2_v, b2_v, exp_v, sem):
    i = pl.program_id(0)

    @pl.when(i == 0)
    def _load_weights():
        for src, dst in ((w1_hbm, w1_v), (w2_hbm, w2_v),
                         (b2_hbm, b2_v), (exp_hbm, exp_v)):
            cp = pltpu.make_async_copy(src, dst, sem)
            cp.start()
            cp.wait()

    E = b2_v.shape[0]
    eids = jax.lax.broadcasted_iota(jnp.int32, (1, E), 1)
    g = gid_ref[...]
    onehot = jnp.where(g == eids, 1.0, 0.0).astype(jnp.bfloat16)
    SUB = 512
    for s in range(_BLK // SUB):
        sl = slice(s * SUB, (s + 1) * SUB)
        oh = onehot[sl, :]
        x = x_ref[sl, :].astype(jnp.bfloat16)
        hpre = jnp.dot(x, w1_v[...], preferred_element_type=jnp.float32)
        bm = jnp.dot(oh, exp_v[...], preferred_element_type=jnp.float32)
        h = jnp.maximum(hpre + bm, 0.0)
        b2sel = jnp.dot(oh, b2_v[...],
                        preferred_element_type=jnp.float32)
        hb = h.astype(jnp.bfloat16)
        out = jnp.dot(hb, w2_v[...], preferred_element_type=jnp.float32)
        o_ref[sl, :] = out + b2sel


def kernel(obs, group_ids, W1, b1, W2, b2, ph_to_feature):
    n_threads, n_agents, d = obs.shape
    E, dp1, H = W1.shape
    A = W2.shape[2]
    T = n_threads * n_agents
    nb = T // _BLK
    EH = E * H

    x = obs.reshape(T, d)
    gid2 = group_ids.reshape(T, 1)
    b1eff = (b1 + ph_to_feature * W1[:, d, :]).reshape(1, EH)
    W1all = jnp.transpose(W1[:, :d, :], (1, 0, 2)).reshape(d, EH)
    W1all = W1all.astype(jnp.bfloat16)
    W2all = W2.reshape(EH, A).astype(jnp.bfloat16)
    b2b = b2.astype(jnp.bfloat16)
    col_e = (jnp.arange(EH, dtype=jnp.int32) // H)[None, :]
    row_e = jnp.arange(E, dtype=jnp.int32)[:, None]
    biasmask = jnp.where(row_e == col_e, b1eff, -1e30).astype(jnp.bfloat16)

    any_spec = pl.BlockSpec(memory_space=pl.ANY)
    out = pl.pallas_call(
        _body,
        grid=(nb,),
        in_specs=[
            pl.BlockSpec((_BLK, 1), lambda i: (i, 0)),
            pl.BlockSpec((_BLK, d), lambda i: (i, 0)),
            any_spec, any_spec, any_spec, any_spec,
        ],
        out_specs=pl.BlockSpec((_BLK, A), lambda i: (i, 0)),
        out_shape=jax.ShapeDtypeStruct((T, A), jnp.float32),
        scratch_shapes=[
            pltpu.VMEM((d, EH), jnp.bfloat16),
            pltpu.VMEM((EH, A), jnp.bfloat16),
            pltpu.VMEM((E, A), jnp.bfloat16),
            pltpu.VMEM((E, EH), jnp.bfloat16),
            pltpu.SemaphoreType.DMA,
        ],
        compiler_params=pltpu.CompilerParams(
            dimension_semantics=("arbitrary",),
        ),
    )(gid2, x, W1all, W2all, b2b, biasmask)
    return out.reshape(n_threads, n_agents, A)
